# baseline (speedup 1.0000x reference)
import functools

import jax
import jax.numpy as jnp
from jax import lax
from jax.experimental import pallas as pl
from jax.experimental.pallas import tpu as pltpu
from jax.experimental.pallas import tpu_sc as plsc

BATCH = 16384
SEQ = 200
HIDDEN = 128
VOCAB = 100
NTOK = BATCH * SEQ

NC = 2
NS = 16
NW = NC * NS
PER_W = NTOK // NW

CHUNK = 128
NCHUNK = PER_W // CHUNK
QUAD = 4
NQUAD = NCHUNK // QUAD

ONES_LEN = 2048

_mesh = plsc.VectorSubcoreMesh(core_axis_name="c", subcore_axis_name="s")


@functools.partial(
    pl.kernel,
    out_type=(
        jax.ShapeDtypeStruct((NTOK, HIDDEN), jnp.float32),
        jax.ShapeDtypeStruct((NTOK,), jnp.float32),
    ),
    mesh=_mesh,
    scratch_types=[
        pltpu.VMEM((QUAD, 1, CHUNK), jnp.int32),
        pltpu.VMEM((QUAD, CHUNK, HIDDEN), jnp.float32),
        pltpu.VMEM((ONES_LEN,), jnp.float32),
        pltpu.VMEM_SHARED((VOCAB, HIDDEN), jnp.float32),
        pltpu.SemaphoreType.DMA,
        pltpu.SemaphoreType.DMA,
        pltpu.SemaphoreType.DMA,
        pltpu.SemaphoreType.DMA,
        pltpu.SemaphoreType.DMA,
        pltpu.SemaphoreType.DMA,
    ],
)
def _embed_sc(tok_hbm, table_hbm, out_hbm, mask_hbm,
              idx_v, rows_v, ones_v, table_sp,
              gsemh, gsems, ssem0, ssem1, ssem2, ssem3):
    ssems = (ssem0, ssem1, ssem2, ssem3)
    sid = lax.axis_index("s")
    wid = sid * NC + lax.axis_index("c")
    base = wid * PER_W
    base_rows = wid * NCHUNK

    @pl.when(sid == 0)
    def _():
        pltpu.sync_copy(table_hbm, table_sp)

    plsc.subcore_barrier()

    def load_idx(c, k):
        pltpu.sync_copy(tok_hbm.at[pl.ds(base_rows + c, 1)], idx_v.at[k])

    def fire_gather(src_table, k, gs):
        return pltpu.async_copy(src_table.at[idx_v.at[k].at[0]], rows_v.at[k], gs)

    def fire_store(c, k):
        pltpu.async_copy(rows_v.at[k], out_hbm.at[pl.ds(base + c * CHUNK, CHUNK)],
                         ssems[k])

    def wait_store(c, k):
        pltpu.make_async_copy(
            rows_v.at[k], out_hbm.at[pl.ds(base + c * CHUNK, CHUNK)], ssems[k]
        ).wait()

    def quad(t, first):
        c0 = QUAD * t
        if not first:
            wait_store(c0 + 3 - QUAD, 3)
        load_idx(c0 + 3, 3)
        hdesc = fire_gather(table_hbm, 3, gsemh)
        for k in range(3):
            if not first:
                wait_store(c0 + k - QUAD, k)
            load_idx(c0 + k, k)
            fire_gather(table_sp, k, gsems).wait()
            fire_store(c0 + k, k)
        hdesc.wait()
        fire_store(c0 + 3, 3)

    quad(0, True)

    def quad_body(t, carry):
        quad(t, False)
        return carry

    lax.fori_loop(1, NQUAD, quad_body, 0)

    def fill_body(i, carry):
        ones_v[pl.ds(i * 16, 16)] = jnp.ones((16,), jnp.float32)
        return carry

    lax.fori_loop(0, ONES_LEN // 16, fill_body, 0)

    def mask_body(g, carry):
        pltpu.sync_copy(ones_v, mask_hbm.at[pl.ds(base + g * ONES_LEN, ONES_LEN)])
        return carry

    lax.fori_loop(0, PER_W // ONES_LEN, mask_body, 0)

    for k in range(QUAD):
        wait_store(QUAD * (NQUAD - 1) + k, k)


def kernel(tokens, table):
    out_flat, mask_flat = _embed_sc(tokens.reshape(NTOK // CHUNK, CHUNK), table)
    return (
        out_flat.reshape(BATCH, SEQ, HIDDEN),
        mask_flat.reshape(BATCH, SEQ),
    )


# speedup vs baseline: 2.6353x; 2.6353x over previous
import functools

import jax
import jax.numpy as jnp
from jax import lax
from jax.experimental import pallas as pl
from jax.experimental.pallas import tpu as pltpu
from jax.experimental.pallas import tpu_sc as plsc

BATCH = 16384
SEQ = 200
HIDDEN = 128
VOCAB = 100
NTOK = BATCH * SEQ

NC = 2
NS = 16
NW = NC * NS
PER_W = NTOK // NW

IDXW = 128
CHUNK = 256
GPC = CHUNK // IDXW
NCHUNK = PER_W // CHUNK
NPAIR = NCHUNK // 2
BLK = 8

ONES_LEN = 2048

_mesh = plsc.VectorSubcoreMesh(core_axis_name="c", subcore_axis_name="s")


@functools.partial(
    pl.kernel,
    out_type=(
        jax.ShapeDtypeStruct((NTOK, HIDDEN), jnp.float32),
        jax.ShapeDtypeStruct((NTOK,), jnp.float32),
    ),
    mesh=_mesh,
    scratch_types=[
        pltpu.VMEM((BLK, GPC, IDXW), jnp.int32),
        pltpu.VMEM((CHUNK, HIDDEN), jnp.float32),
        pltpu.VMEM((CHUNK, HIDDEN), jnp.float32),
        pltpu.VMEM((ONES_LEN,), jnp.float32),
        pltpu.VMEM_SHARED((VOCAB, HIDDEN), jnp.float32),
        pltpu.SemaphoreType.DMA,
        pltpu.SemaphoreType.DMA,
        pltpu.SemaphoreType.DMA,
        pltpu.SemaphoreType.DMA,
    ],
)
def _embed_sc(tok_hbm, table_hbm, out_hbm, mask_hbm,
              idx_blk, rows0, rows1, ones_v, table_sp,
              gsem0, gsem1, ssem0, ssem1):
    sid = lax.axis_index("s")
    wid = sid * NC + lax.axis_index("c")
    base = wid * PER_W
    base_chunks = wid * NCHUNK

    @pl.when(sid == 0)
    def _():
        pltpu.sync_copy(table_hbm, table_sp)

    plsc.subcore_barrier()

    def load_idx_block(g):
        pltpu.sync_copy(tok_hbm.at[pl.ds(base_chunks + g, BLK)], idx_blk)

    def gather_chunk(kk, rows_s, gs_s):
        descs = [
            pltpu.async_copy(
                table_sp.at[idx_blk.at[kk].at[j]],
                rows_s.at[pl.ds(j * IDXW, IDXW)],
                gs_s,
            )
            for j in range(GPC)
        ]
        for d in descs:
            d.wait()

    def fire_store(g, rows_s, ss_s):
        pltpu.async_copy(rows_s, out_hbm.at[pl.ds(base + g * CHUNK, CHUNK)], ss_s)

    def wait_store(g, rows_s, ss_s):
        pltpu.make_async_copy(
            rows_s, out_hbm.at[pl.ds(base + g * CHUNK, CHUNK)], ss_s
        ).wait()

    load_idx_block(0)
    gather_chunk(0, rows0, gsem0)
    fire_store(0, rows0, ssem0)
    gather_chunk(1, rows1, gsem1)
    fire_store(1, rows1, ssem1)

    def pair_body(t, carry):
        g = 2 * t
        kk = lax.rem(g, BLK)

        @pl.when(kk == 0)
        def _():
            load_idx_block(g)

        wait_store(g - 2, rows0, ssem0)
        gather_chunk(kk, rows0, gsem0)
        fire_store(g, rows0, ssem0)
        wait_store(g - 1, rows1, ssem1)
        gather_chunk(kk + 1, rows1, gsem1)
        fire_store(g + 1, rows1, ssem1)
        return carry

    lax.fori_loop(1, NPAIR, pair_body, 0)

    wait_store(NCHUNK - 2, rows0, ssem0)
    wait_store(NCHUNK - 1, rows1, ssem1)

    def fill_body(i, carry):
        ones_v[pl.ds(i * 16, 16)] = jnp.ones((16,), jnp.float32)
        return carry

    lax.fori_loop(0, ONES_LEN // 16, fill_body, 0)

    def mask_body(g, carry):
        pltpu.sync_copy(ones_v, mask_hbm.at[pl.ds(base + g * ONES_LEN, ONES_LEN)])
        return carry

    lax.fori_loop(0, PER_W // ONES_LEN, mask_body, 0)


def kernel(tokens, table):
    out_flat, mask_flat = _embed_sc(
        tokens.reshape(NTOK // CHUNK, GPC, IDXW), table)
    return (
        out_flat.reshape(BATCH, SEQ, HIDDEN),
        mask_flat.reshape(BATCH, SEQ),
    )


# speedup vs baseline: 2.6921x; 1.0216x over previous
import functools

import jax
import jax.numpy as jnp
from jax import lax
from jax.experimental import pallas as pl
from jax.experimental.pallas import tpu as pltpu
from jax.experimental.pallas import tpu_sc as plsc

BATCH = 16384
SEQ = 200
HIDDEN = 128
VOCAB = 100
NTOK = BATCH * SEQ

NC = 2
NS = 16
NW = NC * NS
PER_W = NTOK // NW

IDXW = 128
CHUNK = 256
GPC = CHUNK // IDXW
NCHUNK = PER_W // CHUNK
BLK = 40
NBLK = NCHUNK // BLK

ONES_LEN = 2048

_mesh = plsc.VectorSubcoreMesh(core_axis_name="c", subcore_axis_name="s")


@functools.partial(
    pl.kernel,
    out_type=(
        jax.ShapeDtypeStruct((NTOK, HIDDEN), jnp.float32),
        jax.ShapeDtypeStruct((NTOK,), jnp.float32),
    ),
    mesh=_mesh,
    scratch_types=[
        pltpu.VMEM((BLK, GPC, IDXW), jnp.int32),
        pltpu.VMEM((CHUNK, HIDDEN), jnp.float32),
        pltpu.VMEM((CHUNK, HIDDEN), jnp.float32),
        pltpu.VMEM((ONES_LEN,), jnp.float32),
        pltpu.VMEM_SHARED((VOCAB, HIDDEN), jnp.float32),
        pltpu.SemaphoreType.DMA,
        pltpu.SemaphoreType.DMA,
        pltpu.SemaphoreType.DMA,
        pltpu.SemaphoreType.DMA,
    ],
)
def _embed_sc(tok_hbm, table_hbm, out_hbm, mask_hbm,
              idx_blk, rows0, rows1, ones_v, table_sp,
              gsem0, gsem1, ssem0, ssem1):
    sid = lax.axis_index("s")
    wid = sid * NC + lax.axis_index("c")
    base = wid * PER_W
    base_chunks = wid * NCHUNK

    @pl.when(sid == 0)
    def _():
        pltpu.sync_copy(table_hbm, table_sp)

    plsc.subcore_barrier()

    def fire_gather(kk, rows_s, gs_s):
        for j in range(GPC):
            pltpu.async_copy(
                table_sp.at[idx_blk.at[kk].at[j]],
                rows_s.at[pl.ds(j * IDXW, IDXW)],
                gs_s,
            )

    def wait_gather(kk, rows_s, gs_s):
        for j in range(GPC):
            pltpu.make_async_copy(
                table_sp.at[idx_blk.at[kk].at[j]],
                rows_s.at[pl.ds(j * IDXW, IDXW)],
                gs_s,
            ).wait()

    def fire_store(c, rows_s, ss_s):
        pltpu.async_copy(rows_s, out_hbm.at[pl.ds(base + c * CHUNK, CHUNK)], ss_s)

    def wait_store(c, rows_s, ss_s):
        pltpu.make_async_copy(
            rows_s, out_hbm.at[pl.ds(base + c * CHUNK, CHUNK)], ss_s
        ).wait()

    def block(c0, first_block):
        pltpu.sync_copy(tok_hbm.at[pl.ds(base_chunks + c0, BLK)], idx_blk)
        if not first_block:
            wait_store(c0 - 2, rows0, ssem0)
        fire_gather(0, rows0, gsem0)

        def pair_ops(c, kk, skip_wait1):
            if not skip_wait1:
                wait_store(c - 1, rows1, ssem1)
            fire_gather(kk + 1, rows1, gsem1)
            wait_gather(kk, rows0, gsem0)
            fire_store(c, rows0, ssem0)
            wait_store(c, rows0, ssem0)
            fire_gather(kk + 2, rows0, gsem0)
            wait_gather(kk + 1, rows1, gsem1)
            fire_store(c + 1, rows1, ssem1)

        if first_block:
            pair_ops(c0, 0, True)

        def pair(p, carry):
            pair_ops(c0 + 2 * p, 2 * p, False)
            return carry

        lax.fori_loop(1 if first_block else 0, BLK // 2 - 1, pair, 0)

        c = c0 + BLK - 2
        kk = BLK - 2
        wait_store(c - 1, rows1, ssem1)
        fire_gather(kk + 1, rows1, gsem1)
        wait_gather(kk, rows0, gsem0)
        fire_store(c, rows0, ssem0)
        wait_gather(kk + 1, rows1, gsem1)
        fire_store(c + 1, rows1, ssem1)

    block(0, True)

    def block_body(b, carry):
        block(b * BLK, False)
        return carry

    lax.fori_loop(1, NBLK, block_body, 0)

    def fill_body(i, carry):
        ones_v[pl.ds(i * 16, 16)] = jnp.ones((16,), jnp.float32)
        return carry

    lax.fori_loop(0, ONES_LEN // 16, fill_body, 0)

    def mask_body(g, carry):
        pltpu.sync_copy(ones_v, mask_hbm.at[pl.ds(base + g * ONES_LEN, ONES_LEN)])
        return carry

    lax.fori_loop(0, PER_W // ONES_LEN, mask_body, 0)

    wait_store(NCHUNK - 2, rows0, ssem0)
    wait_store(NCHUNK - 1, rows1, ssem1)


def kernel(tokens, table):
    out_flat, mask_flat = _embed_sc(
        tokens.reshape(NTOK // CHUNK, GPC, IDXW), table)
    return (
        out_flat.reshape(BATCH, SEQ, HIDDEN),
        mask_flat.reshape(BATCH, SEQ),
    )


# speedup vs baseline: 2.8315x; 1.0518x over previous
import functools

import jax
import jax.numpy as jnp
from jax import lax
from jax.experimental import pallas as pl
from jax.experimental.pallas import tpu as pltpu
from jax.experimental.pallas import tpu_sc as plsc

BATCH = 16384
SEQ = 200
HIDDEN = 128
VOCAB = 100
NTOK = BATCH * SEQ

NC = 2
NS = 16
NW = NC * NS
PER_W = NTOK // NW

IDXW = 128
CHUNK = 256
GPC = CHUNK // IDXW
NCHUNK = PER_W // CHUNK
BLK = 40
NBLK = NCHUNK // BLK

MASK_BLOCK = 1024

_mesh = plsc.VectorSubcoreMesh(core_axis_name="c", subcore_axis_name="s")


def _mask_body(o_ref):
    o_ref[...] = jnp.ones_like(o_ref)


_mask_tc = pl.pallas_call(
    _mask_body,
    out_shape=jax.ShapeDtypeStruct((BATCH, SEQ), jnp.float32),
    grid=(BATCH // MASK_BLOCK,),
    out_specs=pl.BlockSpec((MASK_BLOCK, SEQ), lambda i: (i, 0)),
)


@functools.partial(
    pl.kernel,
    out_type=jax.ShapeDtypeStruct((NTOK, HIDDEN), jnp.float32),
    mesh=_mesh,
    scratch_types=[
        pltpu.VMEM((BLK, GPC, IDXW), jnp.int32),
        pltpu.VMEM((CHUNK, HIDDEN), jnp.float32),
        pltpu.VMEM((CHUNK, HIDDEN), jnp.float32),
        pltpu.VMEM_SHARED((VOCAB, HIDDEN), jnp.float32),
        pltpu.SemaphoreType.DMA,
        pltpu.SemaphoreType.DMA,
        pltpu.SemaphoreType.DMA,
        pltpu.SemaphoreType.DMA,
    ],
)
def _embed_sc(tok_hbm, table_hbm, out_hbm,
              idx_blk, rows0, rows1, table_sp,
              gsem0, gsem1, ssem0, ssem1):
    sid = lax.axis_index("s")
    wid = sid * NC + lax.axis_index("c")
    base = wid * PER_W
    base_chunks = wid * NCHUNK

    @pl.when(sid == 0)
    def _():
        pltpu.sync_copy(table_hbm, table_sp)

    plsc.subcore_barrier()

    def fire_gather(kk, rows_s, gs_s):
        for j in range(GPC):
            pltpu.async_copy(
                table_sp.at[idx_blk.at[kk].at[j]],
                rows_s.at[pl.ds(j * IDXW, IDXW)],
                gs_s,
            )

    def wait_gather(kk, rows_s, gs_s):
        for j in range(GPC):
            pltpu.make_async_copy(
                table_sp.at[idx_blk.at[kk].at[j]],
                rows_s.at[pl.ds(j * IDXW, IDXW)],
                gs_s,
            ).wait()

    def fire_store(c, rows_s, ss_s):
        pltpu.async_copy(rows_s, out_hbm.at[pl.ds(base + c * CHUNK, CHUNK)], ss_s)

    def wait_store(c, rows_s, ss_s):
        pltpu.make_async_copy(
            rows_s, out_hbm.at[pl.ds(base + c * CHUNK, CHUNK)], ss_s
        ).wait()

    def block(c0, first_block):
        pltpu.sync_copy(tok_hbm.at[pl.ds(base_chunks + c0, BLK)], idx_blk)
        if not first_block:
            wait_store(c0 - 2, rows0, ssem0)
        fire_gather(0, rows0, gsem0)

        def pair_ops(c, kk, skip_wait1):
            if not skip_wait1:
                wait_store(c - 1, rows1, ssem1)
            fire_gather(kk + 1, rows1, gsem1)
            wait_gather(kk, rows0, gsem0)
            fire_store(c, rows0, ssem0)
            wait_store(c, rows0, ssem0)
            fire_gather(kk + 2, rows0, gsem0)
            wait_gather(kk + 1, rows1, gsem1)
            fire_store(c + 1, rows1, ssem1)

        if first_block:
            pair_ops(c0, 0, True)

        def pair(p, carry):
            pair_ops(c0 + 2 * p, 2 * p, False)
            return carry

        lax.fori_loop(1 if first_block else 0, BLK // 2 - 1, pair, 0)

        c = c0 + BLK - 2
        kk = BLK - 2
        wait_store(c - 1, rows1, ssem1)
        fire_gather(kk + 1, rows1, gsem1)
        wait_gather(kk, rows0, gsem0)
        fire_store(c, rows0, ssem0)
        wait_gather(kk + 1, rows1, gsem1)
        fire_store(c + 1, rows1, ssem1)

    block(0, True)

    def block_body(b, carry):
        block(b * BLK, False)
        return carry

    lax.fori_loop(1, NBLK, block_body, 0)

    wait_store(NCHUNK - 2, rows0, ssem0)
    wait_store(NCHUNK - 1, rows1, ssem1)


def kernel(tokens, table):
    out_flat = _embed_sc(tokens.reshape(NTOK // CHUNK, GPC, IDXW), table)
    return (out_flat.reshape(BATCH, SEQ, HIDDEN), _mask_tc())


# speedup vs baseline: 2.8901x; 1.0207x over previous
import functools

import jax
import jax.numpy as jnp
from jax import lax
from jax.experimental import pallas as pl
from jax.experimental.pallas import tpu as pltpu
from jax.experimental.pallas import tpu_sc as plsc

BATCH = 16384
SEQ = 200
HIDDEN = 128
VOCAB = 100
NTOK = BATCH * SEQ

NC = 2
NS = 16
NW = NC * NS
PER_W = NTOK // NW

IDXW = 128
CHUNK = 256
GPC = CHUNK // IDXW
NCHUNK = PER_W // CHUNK
BLK = 200
NBLK = NCHUNK // BLK

MASK_BLOCK = 1024

_mesh = plsc.VectorSubcoreMesh(core_axis_name="c", subcore_axis_name="s")


def _mask_body(o_ref):
    o_ref[...] = jnp.ones_like(o_ref)


_mask_tc = pl.pallas_call(
    _mask_body,
    out_shape=jax.ShapeDtypeStruct((BATCH, SEQ), jnp.float32),
    grid=(BATCH // MASK_BLOCK,),
    out_specs=pl.BlockSpec((MASK_BLOCK, SEQ), lambda i: (i, 0)),
)


@functools.partial(
    pl.kernel,
    out_type=jax.ShapeDtypeStruct((NTOK, HIDDEN), jnp.float32),
    mesh=_mesh,
    scratch_types=[
        pltpu.VMEM((BLK, GPC, IDXW), jnp.int32),
        pltpu.VMEM((CHUNK, HIDDEN), jnp.float32),
        pltpu.VMEM((CHUNK, HIDDEN), jnp.float32),
        pltpu.VMEM_SHARED((VOCAB, HIDDEN), jnp.float32),
        pltpu.SemaphoreType.DMA,
        pltpu.SemaphoreType.DMA,
        pltpu.SemaphoreType.DMA,
        pltpu.SemaphoreType.DMA,
    ],
)
def _embed_sc(tok_hbm, table_hbm, out_hbm,
              idx_blk, rows0, rows1, table_sp,
              gsem0, gsem1, ssem0, ssem1):
    sid = lax.axis_index("s")
    wid = sid * NC + lax.axis_index("c")
    base = wid * PER_W
    base_chunks = wid * NCHUNK

    @pl.when(sid == 0)
    def _():
        pltpu.sync_copy(table_hbm, table_sp)

    plsc.subcore_barrier()

    def fire_gather(kk, rows_s, gs_s):
        for j in range(GPC):
            pltpu.async_copy(
                table_sp.at[idx_blk.at[kk].at[j]],
                rows_s.at[pl.ds(j * IDXW, IDXW)],
                gs_s,
            )

    def wait_gather(kk, rows_s, gs_s):
        for j in range(GPC):
            pltpu.make_async_copy(
                table_sp.at[idx_blk.at[kk].at[j]],
                rows_s.at[pl.ds(j * IDXW, IDXW)],
                gs_s,
            ).wait()

    def fire_store(c, rows_s, ss_s):
        pltpu.async_copy(rows_s, out_hbm.at[pl.ds(base + c * CHUNK, CHUNK)], ss_s)

    def wait_store(c, rows_s, ss_s):
        pltpu.make_async_copy(
            rows_s, out_hbm.at[pl.ds(base + c * CHUNK, CHUNK)], ss_s
        ).wait()

    def block(c0, first_block):
        pltpu.sync_copy(tok_hbm.at[pl.ds(base_chunks + c0, BLK)], idx_blk)
        if not first_block:
            wait_store(c0 - 2, rows0, ssem0)
        fire_gather(0, rows0, gsem0)

        def pair_ops(c, kk, skip_wait1):
            if not skip_wait1:
                wait_store(c - 1, rows1, ssem1)
            fire_gather(kk + 1, rows1, gsem1)
            wait_gather(kk, rows0, gsem0)
            fire_store(c, rows0, ssem0)
            wait_store(c, rows0, ssem0)
            fire_gather(kk + 2, rows0, gsem0)
            wait_gather(kk + 1, rows1, gsem1)
            fire_store(c + 1, rows1, ssem1)

        if first_block:
            pair_ops(c0, 0, True)

        def pair(p, carry):
            pair_ops(c0 + 2 * p, 2 * p, False)
            return carry

        lax.fori_loop(1 if first_block else 0, BLK // 2 - 1, pair, 0)

        c = c0 + BLK - 2
        kk = BLK - 2
        wait_store(c - 1, rows1, ssem1)
        fire_gather(kk + 1, rows1, gsem1)
        wait_gather(kk, rows0, gsem0)
        fire_store(c, rows0, ssem0)
        wait_gather(kk + 1, rows1, gsem1)
        fire_store(c + 1, rows1, ssem1)

    block(0, True)

    def block_body(b, carry):
        block(b * BLK, False)
        return carry

    lax.fori_loop(1, NBLK, block_body, 0)

    wait_store(NCHUNK - 2, rows0, ssem0)
    wait_store(NCHUNK - 1, rows1, ssem1)


def kernel(tokens, table):
    out_flat = _embed_sc(tokens.reshape(NTOK // CHUNK, GPC, IDXW), table)
    return (out_flat.reshape(BATCH, SEQ, HIDDEN), _mask_tc())


# speedup vs baseline: 2.9182x; 1.0097x over previous
import functools

import jax
import jax.numpy as jnp
from jax import lax
from jax.experimental import pallas as pl
from jax.experimental.pallas import tpu as pltpu
from jax.experimental.pallas import tpu_sc as plsc

BATCH = 16384
SEQ = 200
HIDDEN = 128
VOCAB = 100
NTOK = BATCH * SEQ

NC = 2
NS = 16
NW = NC * NS
PER_W = NTOK // NW

IDXW = 128
CHUNK = 128
NCHUNK = PER_W // CHUNK
NSLOT = 4
BLK = 400
NBLK = NCHUNK // BLK

MASK_BLOCK = 1024

_mesh = plsc.VectorSubcoreMesh(core_axis_name="c", subcore_axis_name="s")


def _mask_body(o_ref):
    o_ref[...] = jnp.ones_like(o_ref)


_mask_tc = pl.pallas_call(
    _mask_body,
    out_shape=jax.ShapeDtypeStruct((BATCH, SEQ), jnp.float32),
    grid=(BATCH // MASK_BLOCK,),
    out_specs=pl.BlockSpec((MASK_BLOCK, SEQ), lambda i: (i, 0)),
)


@functools.partial(
    pl.kernel,
    out_type=jax.ShapeDtypeStruct((NTOK, HIDDEN), jnp.float32),
    mesh=_mesh,
    scratch_types=[
        pltpu.VMEM((BLK, 1, IDXW), jnp.int32),
        pltpu.VMEM((NSLOT, CHUNK, HIDDEN), jnp.float32),
        pltpu.VMEM_SHARED((VOCAB, HIDDEN), jnp.float32),
        pltpu.SemaphoreType.DMA,
        pltpu.SemaphoreType.DMA,
        pltpu.SemaphoreType.DMA,
        pltpu.SemaphoreType.DMA,
        pltpu.SemaphoreType.DMA,
        pltpu.SemaphoreType.DMA,
        pltpu.SemaphoreType.DMA,
        pltpu.SemaphoreType.DMA,
    ],
)
def _embed_sc(tok_hbm, table_hbm, out_hbm,
              idx_blk, rows_v, table_sp,
              gsem0, gsem1, gsem2, gsem3, ssem0, ssem1, ssem2, ssem3):
    gsems = (gsem0, gsem1, gsem2, gsem3)
    ssems = (ssem0, ssem1, ssem2, ssem3)
    sid = lax.axis_index("s")
    wid = sid * NC + lax.axis_index("c")
    base = wid * PER_W
    base_chunks = wid * NCHUNK

    @pl.when(sid == 0)
    def _():
        pltpu.sync_copy(table_hbm, table_sp)

    plsc.subcore_barrier()

    def fire_gather(kk, s):
        pltpu.async_copy(table_sp.at[idx_blk.at[kk].at[0]], rows_v.at[s], gsems[s])

    def wait_gather(kk, s):
        pltpu.make_async_copy(
            table_sp.at[idx_blk.at[kk].at[0]], rows_v.at[s], gsems[s]
        ).wait()

    def fire_store(c, s):
        pltpu.async_copy(rows_v.at[s], out_hbm.at[pl.ds(base + c * CHUNK, CHUNK)],
                         ssems[s])

    def wait_store(c, s):
        pltpu.make_async_copy(
            rows_v.at[s], out_hbm.at[pl.ds(base + c * CHUNK, CHUNK)], ssems[s]
        ).wait()

    def phase(c, kk, s, do_wait, do_fire):
        if do_wait:
            wait_store(c - 3, (s + 1) % NSLOT)
        if do_fire:
            fire_gather(kk + 1, (s + 1) % NSLOT)
        wait_gather(kk, s)
        fire_store(c, s)

    def block(c0, first_block):
        pltpu.sync_copy(tok_hbm.at[pl.ds(base_chunks + c0, BLK)], idx_blk)
        fire_gather(0, 0)

        if first_block:
            for k in range(NSLOT):
                phase(c0 + k, k, k, do_wait=(k == 3), do_fire=True)

        def quad(q, carry):
            cq = c0 + NSLOT * q
            for k in range(NSLOT):
                phase(cq + k, NSLOT * q + k, k, do_wait=True, do_fire=True)
            return carry

        lax.fori_loop(1 if first_block else 0, BLK // NSLOT - 1, quad, 0)

        cq = c0 + BLK - NSLOT
        for k in range(NSLOT):
            phase(cq + k, BLK - NSLOT + k, k, do_wait=True, do_fire=(k < 3))

    block(0, True)

    def block_body(b, carry):
        block(b * BLK, False)
        return carry

    lax.fori_loop(1, NBLK, block_body, 0)

    for k in range(1, NSLOT):
        wait_store(NCHUNK - NSLOT + k, k)


def kernel(tokens, table):
    out_flat = _embed_sc(tokens.reshape(NTOK // CHUNK, 1, IDXW), table)
    return (out_flat.reshape(BATCH, SEQ, HIDDEN), _mask_tc())
